# baseline (speedup 1.0000x reference)
import functools

import jax
import jax.numpy as jnp
from jax import lax
from jax.experimental import pallas as pl
from jax.experimental.pallas import tpu as pltpu
from jax.experimental.pallas import tpu_sc as plsc

_EPS = 1e-07
_W = 64
_S = 100
_NW = 32
_BPW = 128
_SB = 8
_LN2 = 0.6931471805599453


def _prep_body(bwt_ref, mw_ref, dt_ref, aux_ref):
    bwt = bwt_ref[...]
    p = jax.nn.sigmoid(bwt)
    a = jnp.log(p + _EPS)
    c = jnp.log(1.0 - p + _EPS)
    u = 0.5 * jnp.sum(a + c, axis=0)
    mw = mw_ref[0, :]
    mixp = jnp.exp(mw - jnp.max(mw))
    mixp = mixp / jnp.sum(mixp)
    umax = jnp.max(u)
    coef = mixp * jnp.exp(u - umax)
    dt_ref[...] = a - c
    aux_ref[0:_W] = coef
    aux_ref[_W:2 * _W] = jnp.full((_W,), umax, jnp.float32)


def _bcast_lane(v, l, lanes16):
    idx = (lanes16 & 0) + l
    return v.at[idx].get(mode=lax.GatherScatterMode.PROMISE_IN_BOUNDS)


def _lanesum(v, lanes16):
    for sh in (1, 2, 4, 8):
        idx = lanes16 ^ sh
        v = v + v.at[idx].get(mode=lax.GatherScatterMode.PROMISE_IN_BOUNDS)
    return v


def _log16(x):
    xi = lax.bitcast_convert_type(x, jnp.int32)
    e = lax.shift_right_logical(xi, 23) - 127
    m = lax.bitcast_convert_type((xi & 0x007FFFFF) | 0x3F800000,
                                 jnp.float32)
    big = m > (4.0 / 3.0)
    m = jnp.where(big, 0.5 * m, m)
    e = e + jnp.where(big, 1, 0)
    z = m - 1.0
    pz = jnp.float32(-0.125)
    for kk in (7, 6, 5, 4, 3, 2):
        pz = pz * z + ((1.0 / kk) if kk % 2 else (-1.0 / kk))
    pz = z * (1.0 + z * pz)
    return e.astype(jnp.float32) * _LN2 + pz


def _sc_body(s_hbm, dt_hbm, aux_hbm, out_hbm, s_v, dt_v, aux_v, out_v):
    wid = lax.axis_index("s") * 2 + lax.axis_index("c")
    pltpu.sync_copy(s_hbm.at[pl.ds(wid * _BPW, _BPW), :], s_v)
    pltpu.sync_copy(dt_hbm, dt_v)
    pltpu.sync_copy(aux_hbm, aux_v)

    coefs = [aux_v[pl.ds(16 * g, 16)] for g in range(4)]
    uvec = aux_v[pl.ds(_W, 16)]
    zero = jnp.zeros((16,), jnp.float32)
    lanes16 = lax.iota(jnp.int32, 16)

    def mac_lanes(accs, srows, ij, lane):
        dvs = [dt_v[ij, pl.ds(16 * g, 16)] for g in range(4)]
        for b in range(_SB):
            sb = _bcast_lane(srows[b], lane, lanes16)
            for g in range(4):
                accs[b * 4 + g] = accs[b * 4 + g] + sb * dvs[g]
        return accs

    def group_body(grp, _):
        resvec = zero
        for sub in range(2):
            row0 = grp * 16 + sub * _SB

            def chunk_body(c, accs, row0=row0):
                accs = list(accs)
                srows = [s_v[row0 + b, pl.ds(c * 16, 16)] for b in range(_SB)]
                for l in range(16):
                    accs = mac_lanes(accs, srows, c * 16 + l, l)
                return tuple(accs)

            accs = lax.fori_loop(0, _S // 16, chunk_body, (zero,) * (_SB * 4))
            accs = list(accs)
            srows = [s_v[row0 + b, pl.ds(_S - 16, 16)] for b in range(_SB)]
            for l in range(12, 16):
                accs = mac_lanes(accs, srows, _S - 16 + l, l)

            for b in range(_SB):
                acc_e = zero
                for g in range(4):
                    t = 0.5 * accs[b * 4 + g]
                    acc_e = acc_e + coefs[g] * (jnp.exp(t) + jnp.exp(-t))
                e_all = _lanesum(acc_e, lanes16)
                resvec = jnp.where(lanes16 == (sub * _SB + b), e_all, resvec)
        out_v[pl.ds(grp * 16, 16)] = _log16(0.5 * resvec) + uvec
        return 0

    lax.fori_loop(0, _BPW // 16, group_body, 0)
    pltpu.sync_copy(out_v, out_hbm.at[pl.ds(wid * _BPW, _BPW)])


def kernel(sample, ber_weight, mix_weight):
    b = sample.shape[0]
    s2 = sample.reshape(b, _S)
    bwt = ber_weight.reshape(_W, _S).T
    mw2 = mix_weight.reshape(1, _W)

    dt, aux = pl.pallas_call(
        _prep_body,
        out_shape=(jax.ShapeDtypeStruct((_S, _W), jnp.float32),
                   jax.ShapeDtypeStruct((2 * _W,), jnp.float32)),
    )(bwt, mw2)

    sc_main = functools.partial(
        pl.kernel,
        out_type=jax.ShapeDtypeStruct((b,), jnp.float32),
        mesh=plsc.VectorSubcoreMesh(core_axis_name="c", subcore_axis_name="s"),
        scratch_types=[
            pltpu.VMEM((_BPW, _S), jnp.float32),
            pltpu.VMEM((_S, _W), jnp.float32),
            pltpu.VMEM((2 * _W,), jnp.float32),
            pltpu.VMEM((_BPW,), jnp.float32),
        ],
    )(_sc_body)
    return sc_main(s2, dt, aux)


# speedup vs baseline: 4.9333x; 4.9333x over previous
import jax
import jax.numpy as jnp
from jax import lax
from jax.experimental import pallas as pl

_EPS = 1e-07


def _body(st_ref, bw_ref, mw_ref, o_ref):
    bw = bw_ref[...]
    p = jax.nn.sigmoid(bw)
    a = jnp.log(p + _EPS)
    c = jnp.log(1.0 - p + _EPS)
    d = a - c
    u = 0.5 * jnp.sum(a + c, axis=1)
    mw = mw_ref[0, :]
    mixp = jnp.exp(mw - jnp.max(mw))
    mixp = mixp / jnp.sum(mixp)
    umax = jnp.max(u)
    coef = mixp * jnp.exp(u - umax)
    t = 0.5 * lax.dot_general(
        st_ref[...], d, (((0,), (1,)), ((), ())),
        preferred_element_type=jnp.float32)
    e = jnp.exp(t) + jnp.exp(-t)
    acc = jnp.sum(coef[None, :] * e, axis=1)
    o_ref[...] = jnp.log(0.5 * acc) + umax


def kernel(sample, ber_weight, mix_weight):
    b = sample.shape[0]
    st = jnp.transpose(sample, (1, 2, 3, 0)).reshape(-1, b)
    w, s = ber_weight.shape[0], st.shape[0]
    bw2 = ber_weight.reshape(w, s)
    mw2 = mix_weight.reshape(1, w)
    return pl.pallas_call(
        _body,
        out_shape=jax.ShapeDtypeStruct((b,), jnp.float32),
    )(st, bw2, mw2)
